# Initial kernel scaffold; baseline (speedup 1.0000x reference)
import jax
import jax.numpy as jnp
from jax.experimental import pallas as pl


def kernel(story, question, embed_table):
    raise NotImplementedError("write your pallas kernel here")


# speedup vs baseline: 2.7726x; 2.7726x over previous
import jax
import jax.numpy as jnp
from jax import lax
from jax.experimental import pallas as pl
from jax.experimental.pallas import tpu as pltpu
from jax.experimental.pallas import tpu_sc as plsc

EMBED = 64
SRC_LEN = 200
Q_USED = 36
BATCH = 1024
SEQ = Q_USED + SRC_LEN
TOTAL_ROWS = BATCH * SEQ
NC, NS = 2, 16
NW = NC * NS
CHUNK = 128
N_CHUNKS = TOTAL_ROWS // CHUNK
CPW = N_CHUNKS // NW


ROWS_PW = TOTAL_ROWS // NW


def _gather_body(idx_hbm, table_hbm, out_hbm, idx_v, rows_v, sem_g):
    wid = lax.axis_index("s") * NC + lax.axis_index("c")
    r0 = wid * ROWS_PW
    pltpu.sync_copy(idx_hbm.at[pl.ds(r0, ROWS_PW)], idx_v)

    def body(j, carry):
        pltpu.async_copy(
            table_hbm.at[idx_v.at[pl.ds(j * CHUNK, CHUNK)]], rows_v, sem_g
        ).wait()
        pltpu.sync_copy(rows_v, out_hbm.at[pl.ds(r0 + j * CHUNK, CHUNK)])
        return carry

    lax.fori_loop(0, CPW, body, 0)


def kernel(story, question, embed_table):
    idx = jnp.concatenate([question[:Q_USED], story], axis=0)
    idx = idx.astype(jnp.int32).T.reshape(TOTAL_ROWS)

    mesh = plsc.VectorSubcoreMesh(
        core_axis_name="c", subcore_axis_name="s",
        num_cores=NC, num_subcores=NS,
    )
    out = pl.kernel(
        _gather_body,
        out_type=jax.ShapeDtypeStruct((TOTAL_ROWS, EMBED), jnp.float32),
        mesh=mesh,
        scratch_types=[
            pltpu.VMEM((ROWS_PW,), jnp.int32),
            pltpu.VMEM((CHUNK, EMBED), jnp.float32),
            pltpu.SemaphoreType.DMA,
        ],
        compiler_params=pltpu.CompilerParams(use_tc_tiling_on_sc=False),
    )(idx, embed_table)
    return out.reshape(BATCH, SEQ, EMBED)


# speedup vs baseline: 3.1629x; 1.1408x over previous
import jax
import jax.numpy as jnp
from jax import lax
from jax.experimental import pallas as pl
from jax.experimental.pallas import tpu as pltpu
from jax.experimental.pallas import tpu_sc as plsc

EMBED = 64
SRC_LEN = 200
Q_USED = 36
BATCH = 1024
SEQ = Q_USED + SRC_LEN
TOTAL_ROWS = BATCH * SEQ
NC, NS = 2, 16
NW = NC * NS
CHUNK = 128
N_CHUNKS = TOTAL_ROWS // CHUNK
CPW = N_CHUNKS // NW


ROWS_PW = TOTAL_ROWS // NW
BCHUNK = 944
NBCH = ROWS_PW // BCHUNK


def _gather_body(idx_hbm, table_hbm, out_hbm, idx_v, buf0, buf1,
                 sem_g0, sem_g1, sem_w0, sem_w1):
    wid = lax.axis_index("s") * NC + lax.axis_index("c")
    r0 = wid * ROWS_PW
    pltpu.sync_copy(idx_hbm.at[pl.ds(r0, ROWS_PW)], idx_v)

    bufs = (buf0, buf1)
    sem_g = (sem_g0, sem_g1)
    sem_w = (sem_w0, sem_w1)
    gd = [None] * NBCH
    wd = [None] * NBCH

    def start_gather(g):
        gd[g] = pltpu.async_copy(
            table_hbm.at[idx_v.at[pl.ds(g * BCHUNK, BCHUNK)]],
            bufs[g % 2], sem_g[g % 2])

    start_gather(0)
    for g in range(NBCH):
        slot = g % 2
        gd[g].wait()
        if g >= 1:
            wd[g - 1].wait()
        if g < NBCH - 1:
            start_gather(g + 1)
        wd[g] = pltpu.async_copy(
            bufs[slot], out_hbm.at[pl.ds(r0 + g * BCHUNK, BCHUNK)],
            sem_w[slot])
    wd[NBCH - 1].wait()


def kernel(story, question, embed_table):
    idx = jnp.concatenate([question[:Q_USED], story], axis=0)
    idx = idx.astype(jnp.int32).T.reshape(TOTAL_ROWS)

    mesh = plsc.VectorSubcoreMesh(
        core_axis_name="c", subcore_axis_name="s",
        num_cores=NC, num_subcores=NS,
    )
    out = pl.kernel(
        _gather_body,
        out_type=jax.ShapeDtypeStruct((TOTAL_ROWS, EMBED), jnp.float32),
        mesh=mesh,
        scratch_types=[
            pltpu.VMEM((ROWS_PW,), jnp.int32),
            pltpu.VMEM((BCHUNK, EMBED), jnp.float32),
            pltpu.VMEM((BCHUNK, EMBED), jnp.float32),
            pltpu.SemaphoreType.DMA,
            pltpu.SemaphoreType.DMA,
            pltpu.SemaphoreType.DMA,
            pltpu.SemaphoreType.DMA,
        ],
        compiler_params=pltpu.CompilerParams(use_tc_tiling_on_sc=False),
    )(idx, embed_table)
    return out.reshape(BATCH, SEQ, EMBED)
